# baseline (speedup 1.0000x reference)
import functools

import jax
import jax.numpy as jnp
from jax import lax
from jax.experimental import pallas as pl
from jax.experimental.pallas import tpu as pltpu
from jax.experimental.pallas import tpu_sc as plsc


def _tc_body(cat_ref, tab_ref, oe_ref, *, BB, S, L, F, VP, D):
    N = BB * S * L
    cat = cat_ref[...]
    row = lax.broadcasted_iota(jnp.int32, (VP, N), 0)
    emb = jnp.zeros((N, D), jnp.float32)
    for i in range(F):
        ohi = (row == cat[i : i + 1, :]).astype(jnp.bfloat16)
        emb = emb + lax.dot_general(
            ohi,
            tab_ref[i],
            (((0,), (0,)), ((), ())),
            preferred_element_type=jnp.float32,
        )
    oe_ref[...] = emb.reshape(BB, S, L, D)


def _sc_body(cat_hbm, tabs_hbm, rden_hbm, out_hbm, idx_v, rows_v, outb, den_v, sem,
             *, B, S, L, F, DC, D, NW, JPAD):
    wid = lax.axis_index("s") * 2 + lax.axis_index("c")
    bpw = B // NW

    def one_row(t, _):
        bg = wid * bpw + t
        pltpu.sync_copy(cat_hbm.at[bg], idx_v)
        pltpu.sync_copy(rden_hbm.at[bg], den_v)
        copies = []
        for c in range(JPAD // 128):
            copies.append(
                pltpu.async_copy(
                    tabs_hbm.at[idx_v.at[pl.ds(c * 128, 128)]],
                    rows_v.at[pl.ds(c * 128, 128)],
                    sem,
                )
            )
        for cp in copies:
            cp.wait()
        acc = [jnp.zeros((16,), jnp.float32) for _ in range(8)]
        for s in range(S):
            den16 = den_v[s, :]
            for k in range(8):
                outb[s, pl.ds(k * 16, 16)] = acc[k] * den16
            for k in range(8):
                i = k // 2
                off = (k % 2) * 16
                tsum = acc[k]
                for l in range(L):
                    tsum = tsum + rows_v[(s * L + l) * F + i, pl.ds(off, 16)]
                acc[k] = tsum
        pltpu.sync_copy(outb, out_hbm.at[bg])
        return _

    lax.fori_loop(0, bpw, one_row, 0)


def _cumsum(x, axis):
    n = x.shape[axis]
    k = 1
    while k < n:
        pad = jnp.zeros_like(lax.slice_in_dim(x, 0, k, axis=axis))
        x = x + jnp.concatenate(
            [pad, lax.slice_in_dim(x, 0, n - k, axis=axis)], axis=axis
        )
        k *= 2
    return x


def kernel(slates_item_categorical, slates_item_indexes, responses, emb_tables):
    del slates_item_indexes
    B, S, L, F = slates_item_categorical.shape
    _, V, DC = emb_tables.shape
    D = F * DC
    VP = 128
    NW = 32
    JPAD = 1024

    tab = jnp.zeros((F, VP, D), jnp.float32)
    for i in range(F):
        tab = tab.at[i, :V, i * DC : (i + 1) * DC].set(emb_tables[i])
    tabs = jnp.zeros((F * VP, DC), jnp.float32)
    for i in range(F):
        tabs = tabs.at[i * VP : i * VP + V].set(emb_tables[i])
    tab = tab.astype(jnp.bfloat16)

    BB = 16
    grid = B // BB
    N = BB * S * L

    catT = slates_item_categorical.reshape(B * S * L, F).T

    oe = pl.pallas_call(
        functools.partial(_tc_body, BB=BB, S=S, L=L, F=F, VP=VP, D=D),
        grid=(grid,),
        in_specs=[
            pl.BlockSpec((F, N), lambda i: (0, i)),
            pl.BlockSpec((F, VP, D), lambda i: (0, 0, 0)),
        ],
        out_specs=pl.BlockSpec((BB, S, L, D), lambda i: (i, 0, 0, 0)),
        out_shape=jax.ShapeDtypeStruct((B, S, L, D), jnp.float32),
    )(catT, tab)

    offs = (jnp.arange(F, dtype=jnp.int32) * VP)[None, None, :]
    cat3 = slates_item_categorical.reshape(B, S * L, F)
    wmask = (responses > 0).reshape(B, S * L, 1)
    cp = jnp.where(wmask, cat3 + offs, VP - 1).reshape(B, S * L * F)
    catpad = jnp.concatenate(
        [cp, jnp.full((B, JPAD - S * L * F), VP - 1, jnp.int32)], axis=1
    )
    wsum = jnp.sum(responses, axis=2).astype(jnp.float32)
    num = _cumsum(wsum, axis=1) - wsum
    rden = jnp.where(num > 0, 1.0 / jnp.maximum(num, 1.0), 1.0)
    rdenp = jnp.broadcast_to(rden[:, :, None], (B, S, 16))

    sc_cons = functools.partial(
        pl.kernel,
        mesh=plsc.VectorSubcoreMesh(core_axis_name="c", subcore_axis_name="s"),
        out_type=jax.ShapeDtypeStruct((B, S, D), jnp.float32),
        compiler_params=pltpu.CompilerParams(use_tc_tiling_on_sc=False),
        scratch_types=[
            pltpu.VMEM((JPAD,), jnp.int32),
            pltpu.VMEM((JPAD, DC), jnp.float32),
            pltpu.VMEM((S, D), jnp.float32),
            pltpu.VMEM((S, 16), jnp.float32),
            pltpu.SemaphoreType.DMA,
        ],
    )(
        functools.partial(
            _sc_body, B=B, S=S, L=L, F=F, DC=DC, D=D, NW=NW, JPAD=JPAD
        )
    )
    oc2 = sc_cons(catpad, tabs, rdenp)

    return oe, oc2


# speedup vs baseline: 16.4584x; 16.4584x over previous
import functools

import jax
import jax.numpy as jnp
from jax.experimental import pallas as pl


def _cumsum(x, axis):
    n = x.shape[axis]
    k = 1
    while k < n:
        pad = jnp.zeros_like(jax.lax.slice_in_dim(x, 0, k, axis=axis))
        shifted = jnp.concatenate(
            [pad, jax.lax.slice_in_dim(x, 0, n - k, axis=axis)], axis=axis
        )
        x = x + shifted
        k *= 2
    return x


def _body(cat_ref, resp_ref, wrow_ref, q_ref, tab_ref, oe_ref, oc_ref, *, BB, S, L, F, VP, D):
    N = BB * S * L
    cat = cat_ref[...]
    row = jax.lax.broadcasted_iota(jnp.int32, (VP, N), 0)
    emb = jnp.zeros((N, D), jnp.float32)
    for i in range(F):
        ohi = (row == cat[i : i + 1, :]).astype(jnp.bfloat16)
        emb = emb + jax.lax.dot_general(
            ohi,
            tab_ref[i],
            (((0,), (0,)), ((), ())),
            preferred_element_type=jnp.float32,
        )
    oe_ref[...] = emb.reshape(BB, S, L, D)

    wcol = jnp.transpose(wrow_ref[...])
    wemb = (emb * wcol).astype(jnp.bfloat16)
    cons = jnp.dot(q_ref[...], wemb, preferred_element_type=jnp.float32)
    w = resp_ref[...].astype(jnp.float32)
    num = _cumsum(jnp.sum(w, axis=2), axis=1)
    num = num - jnp.sum(w, axis=2)
    denom = jnp.maximum(num, 1.0)[..., None]
    c3 = cons.reshape(BB, S, D)
    oc_ref[...] = jnp.where((num > 0)[..., None], c3 / denom, c3)


def kernel(slates_item_categorical, slates_item_indexes, responses, emb_tables):
    del slates_item_indexes
    B, S, L, F = slates_item_categorical.shape
    _, V, DC = emb_tables.shape
    D = F * DC
    VP = 128

    tab = jnp.zeros((F, VP, D), jnp.float32)
    for i in range(F):
        tab = tab.at[i, :V, i * DC : (i + 1) * DC].set(emb_tables[i])
    tab = tab.astype(jnp.bfloat16)

    BB = 16
    grid = B // BB
    N = BB * S * L

    r = jnp.arange(BB * S, dtype=jnp.int32)
    n = jnp.arange(N, dtype=jnp.int32)
    q = ((n[None, :] // (S * L)) == (r[:, None] // S)) & (
        (n[None, :] % (S * L)) < (r[:, None] % S) * L
    )
    q = q.astype(jnp.bfloat16)

    catT = slates_item_categorical.reshape(B * S * L, F).T
    wrowT = responses.astype(jnp.float32).reshape(1, B * S * L)

    oe, oc = pl.pallas_call(
        functools.partial(_body, BB=BB, S=S, L=L, F=F, VP=VP, D=D),
        grid=(grid,),
        in_specs=[
            pl.BlockSpec((F, N), lambda i: (0, i)),
            pl.BlockSpec((BB, S, L), lambda i: (i, 0, 0)),
            pl.BlockSpec((1, N), lambda i: (0, i)),
            pl.BlockSpec((BB * S, N), lambda i: (0, 0)),
            pl.BlockSpec((F, VP, D), lambda i: (0, 0, 0)),
        ],
        out_specs=[
            pl.BlockSpec((BB, S, L, D), lambda i: (i, 0, 0, 0)),
            pl.BlockSpec((BB, S, D), lambda i: (i, 0, 0)),
        ],
        out_shape=[
            jax.ShapeDtypeStruct((B, S, L, D), jnp.float32),
            jax.ShapeDtypeStruct((B, S, D), jnp.float32),
        ],
    )(catT, responses, wrowT, q, tab)

    return oe, oc
